# baseline (speedup 1.0000x reference)
import functools

import jax
import jax.numpy as jnp
from jax import lax
from jax.experimental import pallas as pl
from jax.experimental.pallas import tpu as pltpu
from jax.experimental.pallas import tpu_sc as plsc

B = 16384
EMB_DIM = 32
BERT_DIM = 768
L = 16
NC, NS = 2, 16
NW = NC * NS
BPW = B // NW
CH = 16
NCHUNK = BPW // CH
EPR = 128 // EMB_DIM
N_EMB_ROWS = 100000 * EMB_DIM // 128
N_BIAS_ROWS = (100000 + 127) // 128
BIAS_PAD = N_BIAS_ROWS * 128 - 100000

_mesh = plsc.VectorSubcoreMesh(core_axis_name="c", subcore_axis_name="s")

_GATHER_DNUMS = lax.GatherDimensionNumbers(
    offset_dims=(), collapsed_slice_dims=(0,), start_index_map=(0,))


def _lane_shuffle(v, idx):
    return lax.gather(v, idx[:, None], _GATHER_DNUMS, (1,),
                      mode=lax.GatherScatterMode.PROMISE_IN_BOUNDS)


@functools.partial(
    pl.kernel,
    out_type=jax.ShapeDtypeStruct((B,), jnp.float32),
    mesh=_mesh,
    compiler_params=pltpu.CompilerParams(needs_layout_passes=False),
    scratch_types=[
        pltpu.VMEM((BPW,), jnp.int32),
        pltpu.VMEM((BPW,), jnp.int32),
        pltpu.VMEM((2, CH), jnp.int32),
        pltpu.VMEM((2, CH), jnp.int32),
        pltpu.VMEM((2, CH), jnp.int32),
        pltpu.VMEM((2, CH), jnp.int32),
        pltpu.VMEM((2, CH, BERT_DIM), jnp.float32),
        pltpu.VMEM((2, CH, BERT_DIM), jnp.float32),
        pltpu.VMEM((2, CH, 128), jnp.float32),
        pltpu.VMEM((2, CH, 128), jnp.float32),
        pltpu.VMEM((2, CH, 128), jnp.float32),
        pltpu.VMEM((2, CH, 128), jnp.float32),
        pltpu.VMEM((BPW,), jnp.float32),
        pltpu.VMEM((L,), jnp.float32),
        pltpu.SemaphoreType.DMA((2,)),
    ],
)
def _sc_kernel(uid_hbm, iid_hbm, uemb_hbm, iemb_hbm, utext_hbm, itext_hbm,
               ubias_hbm, ibias_hbm, bias16_hbm, out_hbm,
               uid_v, iid_v, urow_c, irow_c, ubrow_c, ibrow_c,
               ut_v, it_v, ue_v, ie_v, ub_v, ib_v, out_v, bias_v, sem):
    wid = lax.axis_index("s") * NC + lax.axis_index("c")
    base = wid * BPW

    pltpu.sync_copy(uid_hbm.at[pl.ds(base, BPW)], uid_v)
    pltpu.sync_copy(iid_hbm.at[pl.ds(base, BPW)], iid_v)
    pltpu.sync_copy(bias16_hbm, bias_v)
    bias_vec = bias_v[pl.ds(0, L)]
    lane = lax.iota(jnp.int32, L)
    lane0 = lane == 0

    def issue_chunk(j, p):
        uv = uid_v[pl.ds(j * CH, CH)]
        iv = iid_v[pl.ds(j * CH, CH)]
        urow_c[p, pl.ds(0, CH)] = lax.shift_right_logical(uv, 2)
        irow_c[p, pl.ds(0, CH)] = lax.shift_right_logical(iv, 2)
        ubrow_c[p, pl.ds(0, CH)] = lax.shift_right_logical(uv, 7)
        ibrow_c[p, pl.ds(0, CH)] = lax.shift_right_logical(iv, 7)
        uids = uid_v.at[pl.ds(j * CH, CH)]
        iids = iid_v.at[pl.ds(j * CH, CH)]
        for c in [
            pltpu.make_async_copy(utext_hbm.at[uids], ut_v.at[p], sem.at[p]),
            pltpu.make_async_copy(itext_hbm.at[iids], it_v.at[p], sem.at[p]),
            pltpu.make_async_copy(uemb_hbm.at[urow_c.at[p]], ue_v.at[p],
                                  sem.at[p]),
            pltpu.make_async_copy(iemb_hbm.at[irow_c.at[p]], ie_v.at[p],
                                  sem.at[p]),
            pltpu.make_async_copy(ubias_hbm.at[ubrow_c.at[p]], ub_v.at[p],
                                  sem.at[p]),
            pltpu.make_async_copy(ibias_hbm.at[ibrow_c.at[p]], ib_v.at[p],
                                  sem.at[p]),
        ]:
            c.start()

    def wait_chunk(p):
        for c in [
            pltpu.make_async_copy(utext_hbm.at[urow_c.at[p]], ut_v.at[p],
                                  sem.at[p]),
            pltpu.make_async_copy(itext_hbm.at[irow_c.at[p]], it_v.at[p],
                                  sem.at[p]),
            pltpu.make_async_copy(uemb_hbm.at[urow_c.at[p]], ue_v.at[p],
                                  sem.at[p]),
            pltpu.make_async_copy(iemb_hbm.at[irow_c.at[p]], ie_v.at[p],
                                  sem.at[p]),
            pltpu.make_async_copy(ubias_hbm.at[ubrow_c.at[p]], ub_v.at[p],
                                  sem.at[p]),
            pltpu.make_async_copy(ibias_hbm.at[ibrow_c.at[p]], ib_v.at[p],
                                  sem.at[p]),
        ]:
            c.wait()

    issue_chunk(0, 0)

    def chunk_body(j, carry):
        p = lax.rem(j, 2)
        q = 1 - p

        @pl.when(j < NCHUNK - 1)
        def _issue_next():
            issue_chunk(j + 1, q)

        wait_chunk(p)

        def pair_body(i, carry2):
            pos_in_grp = jnp.broadcast_to(i, (L,))
            uid_b = _lane_shuffle(uid_v[pl.ds(j * CH, L)], pos_in_grp)
            iid_b = _lane_shuffle(iid_v[pl.ds(j * CH, L)], pos_in_grp)
            uq = (uid_b & (EPR - 1)) * EMB_DIM + lane
            iq = (iid_b & (EPR - 1)) * EMB_DIM + lane
            i_b = jnp.broadcast_to(i, (L,)).astype(jnp.int32)
            p_b = jnp.broadcast_to(p, (L,)).astype(jnp.int32)
            ue0 = plsc.load_gather(ue_v, [p_b, i_b, uq])
            ie0 = plsc.load_gather(ie_v, [p_b, i_b, iq])
            ue1 = plsc.load_gather(ue_v, [p_b, i_b, uq + L])
            ie1 = plsc.load_gather(ie_v, [p_b, i_b, iq + L])
            acc = ue0 * ie0 + ue1 * ie1
            for t in range(BERT_DIM // L):
                acc = acc + (ut_v[p, i, pl.ds(t * L, L)]
                             * it_v[p, i, pl.ds(t * L, L)])
            for sh in (8, 4, 2, 1):
                acc = acc + _lane_shuffle(acc, lane ^ sh)
            pos = jnp.broadcast_to(j * CH + i, (L,)).astype(jnp.int32)
            plsc.store_scatter(out_v, [pos], acc, mask=lane0)
            return carry2

        lax.fori_loop(0, CH, pair_body, 0)

        uid_vec = uid_v[pl.ds(j * CH, L)]
        iid_vec = iid_v[pl.ds(j * CH, L)]
        p_b = jnp.broadcast_to(p, (L,)).astype(jnp.int32)
        ubv = plsc.load_gather(ub_v, [p_b, lane, uid_vec & 127])
        ibv = plsc.load_gather(ib_v, [p_b, lane, iid_vec & 127])
        off = j * CH
        out_v[pl.ds(off, L)] = (out_v[pl.ds(off, L)] + ubv + ibv + bias_vec)
        return carry

    lax.fori_loop(0, NCHUNK, chunk_body, 0)
    pltpu.sync_copy(out_v, out_hbm.at[pl.ds(base, BPW)])


def kernel(user_ids, item_ids, user_emb_w, item_emb_w, user_text_w,
           item_text_w, user_bias, item_bias, bias):
    uemb2 = user_emb_w.reshape(N_EMB_ROWS, 128)
    iemb2 = item_emb_w.reshape(N_EMB_ROWS, 128)
    ubias2 = jnp.pad(user_bias, (0, BIAS_PAD)).reshape(N_BIAS_ROWS, 128)
    ibias2 = jnp.pad(item_bias, (0, BIAS_PAD)).reshape(N_BIAS_ROWS, 128)
    bias16 = jnp.broadcast_to(bias, (L,))
    out = _sc_kernel(user_ids, item_ids, uemb2, iemb2, user_text_w,
                     item_text_w, ubias2, ibias2, bias16)
    return out[:, None]


# speedup vs baseline: 1.1851x; 1.1851x over previous
import functools

import jax
import jax.numpy as jnp
from jax import lax
from jax.experimental import pallas as pl
from jax.experimental.pallas import tpu as pltpu
from jax.experimental.pallas import tpu_sc as plsc

B = 16384
EMB_DIM = 32
BERT_DIM = 768
L = 16
NC, NS = 2, 16
NW = NC * NS
BPW = B // NW
EPR = 128 // EMB_DIM
N_EMB_ROWS = 100000 * EMB_DIM // 128

CHT = 32
NCHT = BPW // CHT
CHE = 64
NCHE = BPW // CHE

_GATHER_DNUMS = lax.GatherDimensionNumbers(
    offset_dims=(), collapsed_slice_dims=(0,), start_index_map=(0,))


def _lane_shuffle(v, idx):
    return lax.gather(v, idx[:, None], _GATHER_DNUMS, (1,),
                      mode=lax.GatherScatterMode.PROMISE_IN_BOUNDS)


def _mesh():
    return plsc.VectorSubcoreMesh(core_axis_name="c", subcore_axis_name="s")


def _load_ids(uidf_hbm, iidf_hbm, uidf_v, iidf_v, base):
    pltpu.sync_copy(uidf_hbm.at[pl.ds(base, BPW)], uidf_v)
    pltpu.sync_copy(iidf_hbm.at[pl.ds(base, BPW)], iidf_v)


def _store_i32(dst_ref, ds, value):
    dst_ref[ds] = value


@functools.partial(
    pl.kernel,
    out_type=jax.ShapeDtypeStruct((B,), jnp.float32),
    mesh=_mesh(),
    compiler_params=pltpu.CompilerParams(needs_layout_passes=False),
    scratch_types=[
        pltpu.VMEM((BPW,), jnp.float32),
        pltpu.VMEM((BPW,), jnp.float32),
        pltpu.VMEM((2, CHT), jnp.int32),
        pltpu.VMEM((2, CHT), jnp.int32),
        pltpu.VMEM((2, CHT, BERT_DIM), jnp.float32),
        pltpu.VMEM((2, CHT, BERT_DIM), jnp.float32),
        pltpu.VMEM((BPW,), jnp.float32),
        pltpu.SemaphoreType.DMA((2,)),
    ],
)
def _sc_text(uidf_hbm, iidf_hbm, utext_hbm, itext_hbm, out_hbm,
             uidf_v, iidf_v, uix_c, iix_c, ut_v, it_v, out_v, sem):
    wid = lax.axis_index("s") * NC + lax.axis_index("c")
    base = wid * BPW
    _load_ids(uidf_hbm, iidf_hbm, uidf_v, iidf_v, base)
    lane = lax.iota(jnp.int32, L)
    lane0 = lane == 0

    def issue_chunk(j, p):
        for g in range(CHT // L):
            ds = pl.ds(g * L, L)
            uix_c[p, ds] = plsc.bitcast(uidf_v[pl.ds(j * CHT + g * L, L)],
                                        jnp.int32)
            iix_c[p, ds] = plsc.bitcast(iidf_v[pl.ds(j * CHT + g * L, L)],
                                        jnp.int32)
        cps = _chunk_cps(p)
        for c in cps:
            c.start()

    def _chunk_cps(p):
        return [
            pltpu.make_async_copy(utext_hbm.at[uix_c.at[p]], ut_v.at[p],
                                  sem.at[p]),
            pltpu.make_async_copy(itext_hbm.at[iix_c.at[p]], it_v.at[p],
                                  sem.at[p]),
        ]

    issue_chunk(0, 0)

    def chunk_body(j, carry):
        p = lax.rem(j, 2)
        q = 1 - p

        @pl.when(j < NCHT - 1)
        def _issue_next():
            issue_chunk(j + 1, q)

        for c in _chunk_cps(p):
            c.wait()

        def pair_body(i, carry2):
            acc = ut_v[p, i, pl.ds(0, L)] * it_v[p, i, pl.ds(0, L)]
            for t in range(1, BERT_DIM // L):
                acc = acc + (ut_v[p, i, pl.ds(t * L, L)]
                             * it_v[p, i, pl.ds(t * L, L)])
            for sh in (8, 4, 2, 1):
                acc = acc + _lane_shuffle(acc, lane ^ sh)
            pos = jnp.broadcast_to(j * CHT + i, (L,)).astype(jnp.int32)
            plsc.store_scatter(out_v, [pos], acc, mask=lane0)
            return carry2

        lax.fori_loop(0, CHT, pair_body, 0)
        return carry

    lax.fori_loop(0, NCHT, chunk_body, 0)
    pltpu.sync_copy(out_v, out_hbm.at[pl.ds(base, BPW)])


@functools.partial(
    pl.kernel,
    out_type=jax.ShapeDtypeStruct((B,), jnp.float32),
    mesh=_mesh(),
    compiler_params=pltpu.CompilerParams(needs_layout_passes=False),
    scratch_types=[
        pltpu.VMEM((BPW,), jnp.float32),
        pltpu.VMEM((BPW,), jnp.float32),
        pltpu.VMEM((BPW,), jnp.int32),
        pltpu.VMEM((BPW,), jnp.int32),
        pltpu.VMEM((2, CHE), jnp.int32),
        pltpu.VMEM((2, CHE), jnp.int32),
        pltpu.VMEM((2, CHE, 128), jnp.float32),
        pltpu.VMEM((2, CHE, 128), jnp.float32),
        pltpu.VMEM((2, CHE), jnp.float32),
        pltpu.VMEM((2, CHE), jnp.float32),
        pltpu.VMEM((BPW,), jnp.float32),
        pltpu.VMEM((L,), jnp.float32),
        pltpu.SemaphoreType.DMA((2,)),
    ],
)
def _sc_emb(uidf_hbm, iidf_hbm, uemb_hbm, iemb_hbm, ubias_hbm, ibias_hbm,
            bias16_hbm, out_hbm,
            uidf_v, iidf_v, uid_v, iid_v, urow_c, irow_c, ue_v, ie_v,
            ub_v, ib_v, out_v, bias_v, sem):
    wid = lax.axis_index("s") * NC + lax.axis_index("c")
    base = wid * BPW
    _load_ids(uidf_hbm, iidf_hbm, uidf_v, iidf_v, base)
    pltpu.sync_copy(bias16_hbm, bias_v)
    bias_vec = bias_v[pl.ds(0, L)]
    lane = lax.iota(jnp.int32, L)
    lane0 = lane == 0

    def ids_body(g, carry):
        ds = pl.ds(g * L, L)
        uid_v[ds] = plsc.bitcast(uidf_v[ds], jnp.int32)
        iid_v[ds] = plsc.bitcast(iidf_v[ds], jnp.int32)
        return carry

    lax.fori_loop(0, BPW // L, ids_body, 0)

    def issue_chunk(j, p):
        for g in range(CHE // L):
            ds = pl.ds(g * L, L)
            urow_c[p, ds] = lax.shift_right_logical(
                uid_v[pl.ds(j * CHE + g * L, L)], 2)
            irow_c[p, ds] = lax.shift_right_logical(
                iid_v[pl.ds(j * CHE + g * L, L)], 2)
        for c in _chunk_cps(j, p):
            c.start()

    def _chunk_cps(j, p):
        uids = uid_v.at[pl.ds(j * CHE, CHE)]
        iids = iid_v.at[pl.ds(j * CHE, CHE)]
        return [
            pltpu.make_async_copy(uemb_hbm.at[urow_c.at[p]], ue_v.at[p],
                                  sem.at[p]),
            pltpu.make_async_copy(iemb_hbm.at[irow_c.at[p]], ie_v.at[p],
                                  sem.at[p]),
            pltpu.make_async_copy(ubias_hbm.at[uids], ub_v.at[p], sem.at[p]),
            pltpu.make_async_copy(ibias_hbm.at[iids], ib_v.at[p], sem.at[p]),
        ]

    issue_chunk(0, 0)

    def chunk_body(j, carry):
        p = lax.rem(j, 2)
        q = 1 - p

        @pl.when(j < NCHE - 1)
        def _issue_next():
            issue_chunk(j + 1, q)

        for c in _chunk_cps(j, p):
            c.wait()

        def pair_body(i, carry2):
            grp = j * CHE + i - lax.rem(i, L)
            pos_in_grp = jnp.broadcast_to(lax.rem(i, L), (L,))
            uid_b = _lane_shuffle(uid_v[pl.ds(grp, L)], pos_in_grp)
            iid_b = _lane_shuffle(iid_v[pl.ds(grp, L)], pos_in_grp)
            uq = (uid_b & (EPR - 1)) * EMB_DIM + lane
            iq = (iid_b & (EPR - 1)) * EMB_DIM + lane
            i_b = jnp.broadcast_to(i, (L,)).astype(jnp.int32)
            p_b = jnp.broadcast_to(p, (L,)).astype(jnp.int32)
            ue0 = plsc.load_gather(ue_v, [p_b, i_b, uq])
            ie0 = plsc.load_gather(ie_v, [p_b, i_b, iq])
            ue1 = plsc.load_gather(ue_v, [p_b, i_b, uq + L])
            ie1 = plsc.load_gather(ie_v, [p_b, i_b, iq + L])
            acc = ue0 * ie0 + ue1 * ie1
            for sh in (8, 4, 2, 1):
                acc = acc + _lane_shuffle(acc, lane ^ sh)
            pos = jnp.broadcast_to(j * CHE + i, (L,)).astype(jnp.int32)
            plsc.store_scatter(out_v, [pos], acc, mask=lane0)
            return carry2

        lax.fori_loop(0, CHE, pair_body, 0)

        for gg in range(CHE // L):
            off = j * CHE + gg * L
            out_v[pl.ds(off, L)] = (out_v[pl.ds(off, L)]
                                    + ub_v[p, pl.ds(gg * L, L)]
                                    + ib_v[p, pl.ds(gg * L, L)] + bias_vec)
        return carry

    lax.fori_loop(0, NCHE, chunk_body, 0)
    pltpu.sync_copy(out_v, out_hbm.at[pl.ds(base, BPW)])


def kernel(user_ids, item_ids, user_emb_w, item_emb_w, user_text_w,
           item_text_w, user_bias, item_bias, bias):
    uidf = lax.bitcast_convert_type(user_ids, jnp.float32)
    iidf = lax.bitcast_convert_type(item_ids, jnp.float32)
    uemb2 = user_emb_w.reshape(N_EMB_ROWS, 128)
    iemb2 = item_emb_w.reshape(N_EMB_ROWS, 128)
    bias16 = jnp.broadcast_to(bias, (L,))
    out_t = _sc_text(uidf, iidf, user_text_w, item_text_w)
    out_e = _sc_emb(uidf, iidf, uemb2, iemb2, user_bias, item_bias, bias16)
    return (out_t + out_e)[:, None]


# speedup vs baseline: 1.1867x; 1.0014x over previous
import functools

import jax
import jax.numpy as jnp
from jax import lax
from jax.experimental import pallas as pl
from jax.experimental.pallas import tpu as pltpu
from jax.experimental.pallas import tpu_sc as plsc

B = 16384
EMB_DIM = 32
BERT_DIM = 768
L = 16
NC, NS = 2, 16
NW = NC * NS
BPW = B // NW
EPR = 128 // EMB_DIM
N_EMB_ROWS = 100000 * EMB_DIM // 128

CHT = 32
NCHT = BPW // CHT
CHE = 64
NCHE = BPW // CHE

_GATHER_DNUMS = lax.GatherDimensionNumbers(
    offset_dims=(), collapsed_slice_dims=(0,), start_index_map=(0,))


def _lane_shuffle(v, idx):
    return lax.gather(v, idx[:, None], _GATHER_DNUMS, (1,),
                      mode=lax.GatherScatterMode.PROMISE_IN_BOUNDS)


def _mesh():
    return plsc.VectorSubcoreMesh(core_axis_name="c", subcore_axis_name="s")


def _load_ids(uidf_hbm, iidf_hbm, uidf_v, iidf_v, wid):
    pltpu.sync_copy(uidf_hbm.at[wid], uidf_v)
    pltpu.sync_copy(iidf_hbm.at[wid], iidf_v)


def _store_i32(dst_ref, ds, value):
    dst_ref[ds] = value


@functools.partial(
    pl.kernel,
    out_type=jax.ShapeDtypeStruct((B,), jnp.float32),
    mesh=_mesh(),
    compiler_params=pltpu.CompilerParams(needs_layout_passes=False),
    scratch_types=[
        pltpu.VMEM((BPW,), jnp.float32),
        pltpu.VMEM((BPW,), jnp.float32),
        pltpu.VMEM((2, CHT), jnp.int32),
        pltpu.VMEM((2, CHT), jnp.int32),
        pltpu.VMEM((2, CHT, BERT_DIM), jnp.float32),
        pltpu.VMEM((2, CHT, BERT_DIM), jnp.float32),
        pltpu.VMEM((BPW,), jnp.float32),
        pltpu.SemaphoreType.DMA((2,)),
    ],
)
def _sc_text(uidf_hbm, iidf_hbm, utext_hbm, itext_hbm, out_hbm,
             uidf_v, iidf_v, uix_c, iix_c, ut_v, it_v, out_v, sem):
    wid = lax.axis_index("s") * NC + lax.axis_index("c")
    base = wid * BPW
    _load_ids(uidf_hbm, iidf_hbm, uidf_v, iidf_v, wid)
    lane = lax.iota(jnp.int32, L)
    lane0 = lane == 0

    def issue_chunk(j, p):
        for g in range(CHT // L):
            ds = pl.ds(g * L, L)
            uix_c[p, ds] = plsc.bitcast(uidf_v[pl.ds(j * CHT + g * L, L)],
                                        jnp.int32)
            iix_c[p, ds] = plsc.bitcast(iidf_v[pl.ds(j * CHT + g * L, L)],
                                        jnp.int32)
        cps = _chunk_cps(p)
        for c in cps:
            c.start()

    def _chunk_cps(p):
        return [
            pltpu.make_async_copy(utext_hbm.at[uix_c.at[p]], ut_v.at[p],
                                  sem.at[p]),
            pltpu.make_async_copy(itext_hbm.at[iix_c.at[p]], it_v.at[p],
                                  sem.at[p]),
        ]

    issue_chunk(0, 0)

    def chunk_body(j, carry):
        p = lax.rem(j, 2)
        q = 1 - p

        @pl.when(j < NCHT - 1)
        def _issue_next():
            issue_chunk(j + 1, q)

        for c in _chunk_cps(p):
            c.wait()

        def pair_body(i, carry2):
            acc = ut_v[p, i, pl.ds(0, L)] * it_v[p, i, pl.ds(0, L)]
            for t in range(1, BERT_DIM // L):
                acc = acc + (ut_v[p, i, pl.ds(t * L, L)]
                             * it_v[p, i, pl.ds(t * L, L)])
            for sh in (8, 4, 2, 1):
                acc = acc + _lane_shuffle(acc, lane ^ sh)
            pos = jnp.broadcast_to(j * CHT + i, (L,)).astype(jnp.int32)
            plsc.store_scatter(out_v, [pos], acc, mask=lane0)
            return carry2

        lax.fori_loop(0, CHT, pair_body, 0)
        return carry

    lax.fori_loop(0, NCHT, chunk_body, 0)
    pltpu.sync_copy(out_v, out_hbm.at[pl.ds(base, BPW)])


@functools.partial(
    pl.kernel,
    out_type=jax.ShapeDtypeStruct((B,), jnp.float32),
    mesh=_mesh(),
    compiler_params=pltpu.CompilerParams(needs_layout_passes=False),
    scratch_types=[
        pltpu.VMEM((BPW,), jnp.float32),
        pltpu.VMEM((BPW,), jnp.float32),
        pltpu.VMEM((BPW,), jnp.int32),
        pltpu.VMEM((BPW,), jnp.int32),
        pltpu.VMEM((2, CHE), jnp.int32),
        pltpu.VMEM((2, CHE), jnp.int32),
        pltpu.VMEM((2, CHE, 128), jnp.float32),
        pltpu.VMEM((2, CHE, 128), jnp.float32),
        pltpu.VMEM((2, CHE), jnp.float32),
        pltpu.VMEM((2, CHE), jnp.float32),
        pltpu.VMEM((BPW,), jnp.float32),
        pltpu.VMEM((L,), jnp.float32),
        pltpu.SemaphoreType.DMA((2,)),
    ],
)
def _sc_emb(uidf_hbm, iidf_hbm, uemb_hbm, iemb_hbm, ubias_hbm, ibias_hbm,
            bias16_hbm, out_hbm,
            uidf_v, iidf_v, uid_v, iid_v, urow_c, irow_c, ue_v, ie_v,
            ub_v, ib_v, out_v, bias_v, sem):
    wid = lax.axis_index("s") * NC + lax.axis_index("c")
    base = wid * BPW
    _load_ids(uidf_hbm, iidf_hbm, uidf_v, iidf_v, wid)
    pltpu.sync_copy(bias16_hbm, bias_v)
    bias_vec = bias_v[pl.ds(0, L)]
    lane = lax.iota(jnp.int32, L)
    lane0 = lane == 0

    def ids_body(g, carry):
        ds = pl.ds(g * L, L)
        uid_v[ds] = plsc.bitcast(uidf_v[ds], jnp.int32)
        iid_v[ds] = plsc.bitcast(iidf_v[ds], jnp.int32)
        return carry

    lax.fori_loop(0, BPW // L, ids_body, 0)

    def issue_chunk(j, p):
        for g in range(CHE // L):
            ds = pl.ds(g * L, L)
            urow_c[p, ds] = lax.shift_right_logical(
                uid_v[pl.ds(j * CHE + g * L, L)], 2)
            irow_c[p, ds] = lax.shift_right_logical(
                iid_v[pl.ds(j * CHE + g * L, L)], 2)
        for c in _chunk_cps(j, p):
            c.start()

    def _chunk_cps(j, p):
        uids = uid_v.at[pl.ds(j * CHE, CHE)]
        iids = iid_v.at[pl.ds(j * CHE, CHE)]
        return [
            pltpu.make_async_copy(uemb_hbm.at[urow_c.at[p]], ue_v.at[p],
                                  sem.at[p]),
            pltpu.make_async_copy(iemb_hbm.at[irow_c.at[p]], ie_v.at[p],
                                  sem.at[p]),
            pltpu.make_async_copy(ubias_hbm.at[uids], ub_v.at[p], sem.at[p]),
            pltpu.make_async_copy(ibias_hbm.at[iids], ib_v.at[p], sem.at[p]),
        ]

    issue_chunk(0, 0)

    def chunk_body(j, carry):
        p = lax.rem(j, 2)
        q = 1 - p

        @pl.when(j < NCHE - 1)
        def _issue_next():
            issue_chunk(j + 1, q)

        for c in _chunk_cps(j, p):
            c.wait()

        def pair_body(i, carry2):
            grp = j * CHE + i - lax.rem(i, L)
            pos_in_grp = jnp.broadcast_to(lax.rem(i, L), (L,))
            uid_b = _lane_shuffle(uid_v[pl.ds(grp, L)], pos_in_grp)
            iid_b = _lane_shuffle(iid_v[pl.ds(grp, L)], pos_in_grp)
            uq = (uid_b & (EPR - 1)) * EMB_DIM + lane
            iq = (iid_b & (EPR - 1)) * EMB_DIM + lane
            i_b = jnp.broadcast_to(i, (L,)).astype(jnp.int32)
            p_b = jnp.broadcast_to(p, (L,)).astype(jnp.int32)
            ue0 = plsc.load_gather(ue_v, [p_b, i_b, uq])
            ie0 = plsc.load_gather(ie_v, [p_b, i_b, iq])
            ue1 = plsc.load_gather(ue_v, [p_b, i_b, uq + L])
            ie1 = plsc.load_gather(ie_v, [p_b, i_b, iq + L])
            acc = ue0 * ie0 + ue1 * ie1
            for sh in (8, 4, 2, 1):
                acc = acc + _lane_shuffle(acc, lane ^ sh)
            pos = jnp.broadcast_to(j * CHE + i, (L,)).astype(jnp.int32)
            plsc.store_scatter(out_v, [pos], acc, mask=lane0)
            return carry2

        lax.fori_loop(0, CHE, pair_body, 0)

        for gg in range(CHE // L):
            off = j * CHE + gg * L
            out_v[pl.ds(off, L)] = (out_v[pl.ds(off, L)]
                                    + ub_v[p, pl.ds(gg * L, L)]
                                    + ib_v[p, pl.ds(gg * L, L)] + bias_vec)
        return carry

    lax.fori_loop(0, NCHE, chunk_body, 0)
    pltpu.sync_copy(out_v, out_hbm.at[pl.ds(base, BPW)])


def kernel(user_ids, item_ids, user_emb_w, item_emb_w, user_text_w,
           item_text_w, user_bias, item_bias, bias):
    uidf = lax.bitcast_convert_type(user_ids, jnp.float32).reshape(NW, BPW)
    iidf = lax.bitcast_convert_type(item_ids, jnp.float32).reshape(NW, BPW)
    uemb2 = user_emb_w.reshape(N_EMB_ROWS, 128)
    iemb2 = item_emb_w.reshape(N_EMB_ROWS, 128)
    bias16 = jnp.broadcast_to(bias, (L,))
    out_t = _sc_text(uidf, iidf, user_text_w, item_text_w)
    out_e = _sc_emb(uidf, iidf, uemb2, iemb2, user_bias, item_bias, bias16)
    return (out_t + out_e)[:, None]


# speedup vs baseline: 1.1899x; 1.0027x over previous
import functools

import jax
import jax.numpy as jnp
from jax import lax
from jax.experimental import pallas as pl
from jax.experimental.pallas import tpu as pltpu
from jax.experimental.pallas import tpu_sc as plsc

B = 16384
EMB_DIM = 32
BERT_DIM = 768
L = 16
NC, NS = 2, 16
NW = NC * NS
BPW = B // NW
EPR = 128 // EMB_DIM
N_EMB_ROWS = 100000 * EMB_DIM // 128

CHT = 32
NCHT = BPW // CHT
CHE = 64
NCHE = BPW // CHE

_GATHER_DNUMS = lax.GatherDimensionNumbers(
    offset_dims=(), collapsed_slice_dims=(0,), start_index_map=(0,))


def _lane_shuffle(v, idx):
    return lax.gather(v, idx[:, None], _GATHER_DNUMS, (1,),
                      mode=lax.GatherScatterMode.PROMISE_IN_BOUNDS)


def _mesh():
    return plsc.VectorSubcoreMesh(core_axis_name="c", subcore_axis_name="s")


def _load_ids(uidf_hbm, iidf_hbm, uidf_v, iidf_v, idq, wid, sem):
    idq[pl.ds(0, L)] = jnp.broadcast_to(wid, (L,)).astype(jnp.int32)
    row = idq.at[pl.ds(0, 1)]
    cu = pltpu.make_async_copy(uidf_hbm.at[row], uidf_v, sem)
    ci = pltpu.make_async_copy(iidf_hbm.at[row], iidf_v, sem)
    cu.start()
    ci.start()
    cu.wait()
    ci.wait()


def _store_i32(dst_ref, ds, value):
    dst_ref[ds] = value


@functools.partial(
    pl.kernel,
    out_type=jax.ShapeDtypeStruct((B,), jnp.float32),
    mesh=_mesh(),
    compiler_params=pltpu.CompilerParams(needs_layout_passes=False),
    scratch_types=[
        pltpu.VMEM((1, BPW), jnp.float32),
        pltpu.VMEM((1, BPW), jnp.float32),
        pltpu.VMEM((L,), jnp.int32),
        pltpu.VMEM((2, CHT), jnp.int32),
        pltpu.VMEM((2, CHT), jnp.int32),
        pltpu.VMEM((2, CHT, BERT_DIM), jnp.float32),
        pltpu.VMEM((2, CHT, BERT_DIM), jnp.float32),
        pltpu.VMEM((BPW,), jnp.float32),
        pltpu.SemaphoreType.DMA((2,)),
    ],
)
def _sc_text(uidf_hbm, iidf_hbm, utext_hbm, itext_hbm, out_hbm,
             uidf_v, iidf_v, idq, uix_c, iix_c, ut_v, it_v, out_v, sem):
    wid = lax.axis_index("s") * NC + lax.axis_index("c")
    base = wid * BPW
    _load_ids(uidf_hbm, iidf_hbm, uidf_v, iidf_v, idq, wid, sem.at[0])
    lane = lax.iota(jnp.int32, L)
    lane0 = lane == 0

    def issue_chunk(j, p):
        for g in range(CHT // L):
            ds = pl.ds(g * L, L)
            uix_c[p, ds] = plsc.bitcast(
                uidf_v[0, pl.ds(j * CHT + g * L, L)], jnp.int32)
            iix_c[p, ds] = plsc.bitcast(
                iidf_v[0, pl.ds(j * CHT + g * L, L)], jnp.int32)
        cps = _chunk_cps(p)
        for c in cps:
            c.start()

    def _chunk_cps(p):
        return [
            pltpu.make_async_copy(utext_hbm.at[uix_c.at[p]], ut_v.at[p],
                                  sem.at[p]),
            pltpu.make_async_copy(itext_hbm.at[iix_c.at[p]], it_v.at[p],
                                  sem.at[p]),
        ]

    issue_chunk(0, 0)

    def chunk_body(j, carry):
        p = lax.rem(j, 2)
        q = 1 - p

        @pl.when(j < NCHT - 1)
        def _issue_next():
            issue_chunk(j + 1, q)

        for c in _chunk_cps(p):
            c.wait()

        def pair_body(i, carry2):
            acc = ut_v[p, i, pl.ds(0, L)] * it_v[p, i, pl.ds(0, L)]
            for t in range(1, BERT_DIM // L):
                acc = acc + (ut_v[p, i, pl.ds(t * L, L)]
                             * it_v[p, i, pl.ds(t * L, L)])
            for sh in (8, 4, 2, 1):
                acc = acc + _lane_shuffle(acc, lane ^ sh)
            pos = jnp.broadcast_to(j * CHT + i, (L,)).astype(jnp.int32)
            plsc.store_scatter(out_v, [pos], acc, mask=lane0)
            return carry2

        lax.fori_loop(0, CHT, pair_body, 0)
        return carry

    lax.fori_loop(0, NCHT, chunk_body, 0)
    pltpu.sync_copy(out_v, out_hbm.at[pl.ds(base, BPW)])


@functools.partial(
    pl.kernel,
    out_type=jax.ShapeDtypeStruct((B,), jnp.float32),
    mesh=_mesh(),
    compiler_params=pltpu.CompilerParams(needs_layout_passes=False),
    scratch_types=[
        pltpu.VMEM((1, BPW), jnp.float32),
        pltpu.VMEM((1, BPW), jnp.float32),
        pltpu.VMEM((L,), jnp.int32),
        pltpu.VMEM((BPW,), jnp.int32),
        pltpu.VMEM((BPW,), jnp.int32),
        pltpu.VMEM((2, CHE), jnp.int32),
        pltpu.VMEM((2, CHE), jnp.int32),
        pltpu.VMEM((2, CHE, 128), jnp.float32),
        pltpu.VMEM((2, CHE, 128), jnp.float32),
        pltpu.VMEM((2, CHE), jnp.float32),
        pltpu.VMEM((2, CHE), jnp.float32),
        pltpu.VMEM((BPW,), jnp.float32),
        pltpu.VMEM((L,), jnp.float32),
        pltpu.SemaphoreType.DMA((2,)),
    ],
)
def _sc_emb(uidf_hbm, iidf_hbm, uemb_hbm, iemb_hbm, ubias_hbm, ibias_hbm,
            bias16_hbm, out_hbm,
            uidf_v, iidf_v, idq, uid_v, iid_v, urow_c, irow_c, ue_v, ie_v,
            ub_v, ib_v, out_v, bias_v, sem):
    wid = lax.axis_index("s") * NC + lax.axis_index("c")
    base = wid * BPW
    _load_ids(uidf_hbm, iidf_hbm, uidf_v, iidf_v, idq, wid, sem.at[0])
    pltpu.sync_copy(bias16_hbm, bias_v)
    bias_vec = bias_v[pl.ds(0, L)]
    lane = lax.iota(jnp.int32, L)
    lane0 = lane == 0

    def ids_body(g, carry):
        ds = pl.ds(g * L, L)
        uid_v[ds] = plsc.bitcast(uidf_v[0, ds], jnp.int32)
        iid_v[ds] = plsc.bitcast(iidf_v[0, ds], jnp.int32)
        return carry

    lax.fori_loop(0, BPW // L, ids_body, 0)

    def issue_chunk(j, p):
        for g in range(CHE // L):
            ds = pl.ds(g * L, L)
            urow_c[p, ds] = lax.shift_right_logical(
                uid_v[pl.ds(j * CHE + g * L, L)], 2)
            irow_c[p, ds] = lax.shift_right_logical(
                iid_v[pl.ds(j * CHE + g * L, L)], 2)
        for c in _chunk_cps(j, p):
            c.start()

    def _chunk_cps(j, p):
        uids = uid_v.at[pl.ds(j * CHE, CHE)]
        iids = iid_v.at[pl.ds(j * CHE, CHE)]
        return [
            pltpu.make_async_copy(uemb_hbm.at[urow_c.at[p]], ue_v.at[p],
                                  sem.at[p]),
            pltpu.make_async_copy(iemb_hbm.at[irow_c.at[p]], ie_v.at[p],
                                  sem.at[p]),
            pltpu.make_async_copy(ubias_hbm.at[uids], ub_v.at[p], sem.at[p]),
            pltpu.make_async_copy(ibias_hbm.at[iids], ib_v.at[p], sem.at[p]),
        ]

    issue_chunk(0, 0)

    def chunk_body(j, carry):
        p = lax.rem(j, 2)
        q = 1 - p

        @pl.when(j < NCHE - 1)
        def _issue_next():
            issue_chunk(j + 1, q)

        for c in _chunk_cps(j, p):
            c.wait()

        def pair_body(i, carry2):
            grp = j * CHE + i - lax.rem(i, L)
            pos_in_grp = jnp.broadcast_to(lax.rem(i, L), (L,))
            uid_b = _lane_shuffle(uid_v[pl.ds(grp, L)], pos_in_grp)
            iid_b = _lane_shuffle(iid_v[pl.ds(grp, L)], pos_in_grp)
            uq = (uid_b & (EPR - 1)) * EMB_DIM + lane
            iq = (iid_b & (EPR - 1)) * EMB_DIM + lane
            i_b = jnp.broadcast_to(i, (L,)).astype(jnp.int32)
            p_b = jnp.broadcast_to(p, (L,)).astype(jnp.int32)
            ue0 = plsc.load_gather(ue_v, [p_b, i_b, uq])
            ie0 = plsc.load_gather(ie_v, [p_b, i_b, iq])
            ue1 = plsc.load_gather(ue_v, [p_b, i_b, uq + L])
            ie1 = plsc.load_gather(ie_v, [p_b, i_b, iq + L])
            acc = ue0 * ie0 + ue1 * ie1
            for sh in (8, 4, 2, 1):
                acc = acc + _lane_shuffle(acc, lane ^ sh)
            pos = jnp.broadcast_to(j * CHE + i, (L,)).astype(jnp.int32)
            plsc.store_scatter(out_v, [pos], acc, mask=lane0)
            return carry2

        lax.fori_loop(0, CHE, pair_body, 0)

        for gg in range(CHE // L):
            off = j * CHE + gg * L
            out_v[pl.ds(off, L)] = (out_v[pl.ds(off, L)]
                                    + ub_v[p, pl.ds(gg * L, L)]
                                    + ib_v[p, pl.ds(gg * L, L)] + bias_vec)
        return carry

    lax.fori_loop(0, NCHE, chunk_body, 0)
    pltpu.sync_copy(out_v, out_hbm.at[pl.ds(base, BPW)])


def kernel(user_ids, item_ids, user_emb_w, item_emb_w, user_text_w,
           item_text_w, user_bias, item_bias, bias):
    uidf = lax.bitcast_convert_type(user_ids, jnp.float32).reshape(NW, BPW)
    iidf = lax.bitcast_convert_type(item_ids, jnp.float32).reshape(NW, BPW)
    uemb2 = user_emb_w.reshape(N_EMB_ROWS, 128)
    iemb2 = item_emb_w.reshape(N_EMB_ROWS, 128)
    bias16 = jnp.broadcast_to(bias, (L,))
    out_t = _sc_text(uidf, iidf, user_text_w, item_text_w)
    out_e = _sc_emb(uidf, iidf, uemb2, iemb2, user_bias, item_bias, bias16)
    return (out_t + out_e)[:, None]
